# Initial kernel scaffold; baseline (speedup 1.0000x reference)
import jax
import jax.numpy as jnp
from jax.experimental import pallas as pl


def kernel(descriptions, amounts, table, W, b):
    raise NotImplementedError("write your pallas kernel here")


# speedup vs baseline: 2.7118x; 2.7118x over previous
import functools

import jax
import jax.numpy as jnp
from jax import lax
from jax.experimental import pallas as pl
from jax.experimental.pallas import tpu as pltpu
from jax.experimental.pallas import tpu_sc as plsc

VOCAB = 1000000
EMBED = 32
NUM_CAT = 128
B = 16384
L = 50

NC = 2
NS = 16
NW = NC * NS
B_PER_W = B // NW
G = 32
N_CHUNKS = B_PER_W // G
IDX_PER_CHUNK = G * L
GATHER_W = 128


def _sc_pool(table, desc_flat):
    mesh = plsc.VectorSubcoreMesh(core_axis_name="c", subcore_axis_name="s")

    @functools.partial(
        pl.kernel,
        out_type=jax.ShapeDtypeStruct((B, EMBED), jnp.float32),
        mesh=mesh,
        compiler_params=pltpu.CompilerParams(use_tc_tiling_on_sc=False),
        scratch_types=[
            pltpu.VMEM((IDX_PER_CHUNK,), jnp.int32),
            pltpu.VMEM((IDX_PER_CHUNK, EMBED), jnp.float32),
            pltpu.VMEM((G, EMBED), jnp.float32),
            pltpu.SemaphoreType.DMA,
        ],
    )
    def pool_kernel(table_hbm, idx_hbm, out_hbm, idx_v, rows_v, acc_v, sem):
        wid = lax.axis_index("s") * NC + lax.axis_index("c")
        item_base = wid * B_PER_W

        @pl.loop(0, N_CHUNKS)
        def _(chunk):
            item0 = item_base + chunk * G
            pltpu.sync_copy(idx_hbm.at[pl.ds(item0 * L, IDX_PER_CHUNK)], idx_v)
            copies = []
            for off in range(0, IDX_PER_CHUNK, GATHER_W):
                w = min(GATHER_W, IDX_PER_CHUNK - off)
                copies.append(pltpu.async_copy(
                    table_hbm.at[idx_v.at[pl.ds(off, w)]],
                    rows_v.at[pl.ds(off, w)],
                    sem,
                ))
            for c in copies:
                c.wait()

            @pl.loop(0, G)
            def _(j):
                r0 = j * L
                lo0 = rows_v[r0, 0:16]
                hi0 = rows_v[r0, 16:32]
                lo1 = rows_v[r0 + 1, 0:16]
                hi1 = rows_v[r0 + 1, 16:32]
                for l in range(2, L, 2):
                    lo0 += rows_v[r0 + l, 0:16]
                    hi0 += rows_v[r0 + l, 16:32]
                    lo1 += rows_v[r0 + l + 1, 0:16]
                    hi1 += rows_v[r0 + l + 1, 16:32]
                acc_v[j, 0:16] = lo0 + lo1
                acc_v[j, 16:32] = hi0 + hi1

            pltpu.sync_copy(acc_v, out_hbm.at[pl.ds(item0, G)])

    return pool_kernel(table, desc_flat)


BLK = 2048


def _tc_body(sums_ref, amounts_ref, w_ref, b_ref, out_ref):
    x = sums_ref[...] * (1.0 / L)
    w0 = w_ref[0:EMBED, :]
    w1 = w_ref[EMBED:EMBED + 1, :]
    out_ref[...] = (
        jnp.dot(x, w0, preferred_element_type=jnp.float32,
                precision=jax.lax.Precision.HIGHEST)
        + amounts_ref[...] * w1
        + b_ref[...]
    )


def _tc_linear(sums, amounts, W, b2d):
    return pl.pallas_call(
        _tc_body,
        grid=(B // BLK,),
        in_specs=[
            pl.BlockSpec((BLK, EMBED), lambda i: (i, 0)),
            pl.BlockSpec((BLK, 1), lambda i: (i, 0)),
            pl.BlockSpec((EMBED + 1, NUM_CAT), lambda i: (0, 0)),
            pl.BlockSpec((1, NUM_CAT), lambda i: (0, 0)),
        ],
        out_specs=pl.BlockSpec((BLK, NUM_CAT), lambda i: (i, 0)),
        out_shape=jax.ShapeDtypeStruct((B, NUM_CAT), jnp.float32),
    )(sums, amounts, W, b2d)


def kernel(descriptions, amounts, table, W, b):
    desc_flat = descriptions.reshape(-1).astype(jnp.int32)
    sums = _sc_pool(table, desc_flat)
    return _tc_linear(sums, amounts, W, b.reshape(1, NUM_CAT))


# speedup vs baseline: 2.8732x; 1.0595x over previous
import functools

import jax
import jax.numpy as jnp
from jax import lax
from jax.experimental import pallas as pl
from jax.experimental.pallas import tpu as pltpu
from jax.experimental.pallas import tpu_sc as plsc

VOCAB = 1000000
EMBED = 32
NUM_CAT = 128
B = 16384
L = 50

NC = 2
NS = 16
NW = NC * NS
B_PER_W = B // NW
G = 32
N_CHUNKS = B_PER_W // G
IDX_PER_CHUNK = G * L
GATHER_W = 128


def _sc_pool(table, desc_flat):
    mesh = plsc.VectorSubcoreMesh(core_axis_name="c", subcore_axis_name="s")

    @functools.partial(
        pl.kernel,
        out_type=jax.ShapeDtypeStruct((B, EMBED), jnp.float32),
        mesh=mesh,
        compiler_params=pltpu.CompilerParams(use_tc_tiling_on_sc=False),
        scratch_types=[
            pltpu.VMEM((IDX_PER_CHUNK,), jnp.int32),
            pltpu.VMEM((IDX_PER_CHUNK,), jnp.int32),
            pltpu.VMEM((IDX_PER_CHUNK, EMBED), jnp.float32),
            pltpu.VMEM((IDX_PER_CHUNK, EMBED), jnp.float32),
            pltpu.VMEM((G, EMBED), jnp.float32),
            pltpu.VMEM((G, EMBED), jnp.float32),
            pltpu.SemaphoreType.DMA,
            pltpu.SemaphoreType.DMA,
            pltpu.SemaphoreType.DMA,
            pltpu.SemaphoreType.DMA,
        ],
    )
    def pool_kernel(table_hbm, idx_hbm, out_hbm,
                    idx_a, idx_b, rows_a, rows_b, acc_a, acc_b,
                    gsem_a, gsem_b, ssem_a, ssem_b):
        wid = lax.axis_index("s") * NC + lax.axis_index("c")
        item_base = wid * B_PER_W
        idx_v = (idx_a, idx_b)
        rows_v = (rows_a, rows_b)
        acc_v = (acc_a, acc_b)
        gsem = (gsem_a, gsem_b)
        ssem = (ssem_a, ssem_b)

        def fire(g):
            p = g % 2
            item0 = item_base + g * G
            pltpu.sync_copy(idx_hbm.at[pl.ds(item0 * L, IDX_PER_CHUNK)], idx_v[p])
            handles = []
            for off in range(0, IDX_PER_CHUNK, GATHER_W):
                w = min(GATHER_W, IDX_PER_CHUNK - off)
                handles.append(pltpu.async_copy(
                    table_hbm.at[idx_v[p].at[pl.ds(off, w)]],
                    rows_v[p].at[pl.ds(off, w)],
                    gsem[p],
                ))
            return handles

        def reduce_store(g):
            p = g % 2
            rows = rows_v[p]
            acc = acc_v[p]

            @pl.loop(0, G)
            def _(j):
                r0 = j * L
                lo0 = rows[r0, 0:16]
                hi0 = rows[r0, 16:32]
                lo1 = rows[r0 + 1, 0:16]
                hi1 = rows[r0 + 1, 16:32]
                for l in range(2, L, 2):
                    lo0 += rows[r0 + l, 0:16]
                    hi0 += rows[r0 + l, 16:32]
                    lo1 += rows[r0 + l + 1, 0:16]
                    hi1 += rows[r0 + l + 1, 16:32]
                acc[j, 0:16] = lo0 + lo1
                acc[j, 16:32] = hi0 + hi1

            item0 = item_base + g * G
            return pltpu.async_copy(acc, out_hbm.at[pl.ds(item0, G)], ssem[p])

        store_handles = [None, None]
        handles = fire(0)
        for g in range(N_CHUNKS):
            nxt = fire(g + 1) if g + 1 < N_CHUNKS else None
            for h in handles:
                h.wait()
            if store_handles[g % 2] is not None:
                store_handles[g % 2].wait()
            store_handles[g % 2] = reduce_store(g)
            handles = nxt
        for sh in store_handles:
            if sh is not None:
                sh.wait()

    return pool_kernel(table, desc_flat)


BLK = 2048


def _tc_body(sums_ref, amounts_ref, w_ref, b_ref, out_ref):
    x = sums_ref[...] * (1.0 / L)
    w0 = w_ref[0:EMBED, :]
    w1 = w_ref[EMBED:EMBED + 1, :]
    out_ref[...] = (
        jnp.dot(x, w0, preferred_element_type=jnp.float32,
                precision=jax.lax.Precision.HIGHEST)
        + amounts_ref[...] * w1
        + b_ref[...]
    )


def _tc_linear(sums, amounts, W, b2d):
    return pl.pallas_call(
        _tc_body,
        grid=(B // BLK,),
        in_specs=[
            pl.BlockSpec((BLK, EMBED), lambda i: (i, 0)),
            pl.BlockSpec((BLK, 1), lambda i: (i, 0)),
            pl.BlockSpec((EMBED + 1, NUM_CAT), lambda i: (0, 0)),
            pl.BlockSpec((1, NUM_CAT), lambda i: (0, 0)),
        ],
        out_specs=pl.BlockSpec((BLK, NUM_CAT), lambda i: (i, 0)),
        out_shape=jax.ShapeDtypeStruct((B, NUM_CAT), jnp.float32),
    )(sums, amounts, W, b2d)


def kernel(descriptions, amounts, table, W, b):
    desc_flat = descriptions.reshape(-1).astype(jnp.int32)
    sums = _sc_pool(table, desc_flat)
    return _tc_linear(sums, amounts, W, b.reshape(1, NUM_CAT))


# speedup vs baseline: 3.0953x; 1.0773x over previous
import functools

import jax
import jax.numpy as jnp
from jax import lax
from jax.experimental import pallas as pl
from jax.experimental.pallas import tpu as pltpu
from jax.experimental.pallas import tpu_sc as plsc

VOCAB = 1000000
EMBED = 32
NUM_CAT = 128
B = 16384
L = 50

NC = 2
NS = 16
NW = NC * NS
B_PER_W = B // NW
G = 32
N_CHUNKS = B_PER_W // G
IDX_PER_CHUNK = G * L
GATHER_W = 128


def _sc_pool(table, desc_flat):
    mesh = plsc.VectorSubcoreMesh(core_axis_name="c", subcore_axis_name="s")

    @functools.partial(
        pl.kernel,
        out_type=jax.ShapeDtypeStruct((B, EMBED), jnp.float32),
        mesh=mesh,
        compiler_params=pltpu.CompilerParams(use_tc_tiling_on_sc=False),
        scratch_types=[
            pltpu.VMEM((IDX_PER_CHUNK,), jnp.int32),
            pltpu.VMEM((IDX_PER_CHUNK,), jnp.int32),
            pltpu.VMEM((IDX_PER_CHUNK, EMBED), jnp.float32),
            pltpu.VMEM((IDX_PER_CHUNK, EMBED), jnp.float32),
            pltpu.VMEM((G, EMBED), jnp.float32),
            pltpu.VMEM((G, EMBED), jnp.float32),
            pltpu.SemaphoreType.DMA,
            pltpu.SemaphoreType.DMA,
            pltpu.SemaphoreType.DMA,
            pltpu.SemaphoreType.DMA,
        ],
    )
    def pool_kernel(table_hbm, idx_hbm, out_hbm,
                    idx_a, idx_b, rows_a, rows_b, acc_a, acc_b,
                    gsem_a, gsem_b, ssem_a, ssem_b):
        wid = lax.axis_index("s") * NC + lax.axis_index("c")
        item_base = wid * B_PER_W
        idx_v = (idx_a, idx_b)
        rows_v = (rows_a, rows_b)
        acc_v = (acc_a, acc_b)
        gsem = (gsem_a, gsem_b)
        ssem = (ssem_a, ssem_b)

        def fire(g):
            p = g % 2
            item0 = item_base + g * G
            pltpu.sync_copy(idx_hbm.at[pl.ds(item0 * L, IDX_PER_CHUNK)], idx_v[p])
            handles = []
            for off in range(0, IDX_PER_CHUNK, GATHER_W):
                w = min(GATHER_W, IDX_PER_CHUNK - off)
                handles.append(pltpu.async_copy(
                    table_hbm.at[idx_v[p].at[pl.ds(off, w)]],
                    rows_v[p].at[pl.ds(off, w)],
                    gsem[p],
                ))
            return handles

        def reduce_store(g):
            p = g % 2
            rows = rows_v[p]
            acc = acc_v[p]

            @pl.loop(0, G)
            def _(j):
                r0 = j * L
                lo0 = rows[r0, 0:16]
                hi0 = rows[r0, 16:32]
                lo1 = rows[r0 + 1, 0:16]
                hi1 = rows[r0 + 1, 16:32]
                for l in range(2, L, 2):
                    lo0 += rows[r0 + l, 0:16]
                    hi0 += rows[r0 + l, 16:32]
                    lo1 += rows[r0 + l + 1, 0:16]
                    hi1 += rows[r0 + l + 1, 16:32]
                acc[j, 0:16] = lo0 + lo1
                acc[j, 16:32] = hi0 + hi1

            item0 = item_base + g * G
            return pltpu.async_copy(acc, out_hbm.at[pl.ds(item0, G)], ssem[p])

        store_handles = [None, None]
        handles = fire(0)
        for g in range(N_CHUNKS):
            nxt = fire(g + 1) if g + 1 < N_CHUNKS else None
            for h in handles:
                h.wait()
            if store_handles[g % 2] is not None:
                store_handles[g % 2].wait()
            store_handles[g % 2] = reduce_store(g)
            handles = nxt
        for sh in store_handles:
            if sh is not None:
                sh.wait()

    return pool_kernel(table, desc_flat)


TBLK = 2048


def _tt_body(tt_ref, out_ref):
    out_ref[:, 0:EMBED] = jnp.swapaxes(tt_ref[...], 0, 1)


def _tc_transpose(table_t):
    return pl.pallas_call(
        _tt_body,
        grid=(pl.cdiv(VOCAB, TBLK),),
        in_specs=[pl.BlockSpec((EMBED, TBLK), lambda i: (0, i))],
        out_specs=pl.BlockSpec((TBLK, 128), lambda i: (i, 0)),
        out_shape=jax.ShapeDtypeStruct((VOCAB, 128), jnp.float32),
    )(table_t)


BLK = 2048


def _tc_body(sums_ref, amounts_ref, w_ref, b_ref, out_ref):
    x = sums_ref[...] * (1.0 / L)
    w0 = w_ref[0:EMBED, :]
    w1 = w_ref[EMBED:EMBED + 1, :]
    out_ref[...] = (
        jnp.dot(x, w0, preferred_element_type=jnp.float32,
                precision=jax.lax.Precision.HIGHEST)
        + amounts_ref[...] * w1
        + b_ref[...]
    )


def _tc_linear(sums, amounts, W, b2d):
    return pl.pallas_call(
        _tc_body,
        grid=(B // BLK,),
        in_specs=[
            pl.BlockSpec((BLK, EMBED), lambda i: (i, 0)),
            pl.BlockSpec((BLK, 1), lambda i: (i, 0)),
            pl.BlockSpec((EMBED + 1, NUM_CAT), lambda i: (0, 0)),
            pl.BlockSpec((1, NUM_CAT), lambda i: (0, 0)),
        ],
        out_specs=pl.BlockSpec((BLK, NUM_CAT), lambda i: (i, 0)),
        out_shape=jax.ShapeDtypeStruct((B, NUM_CAT), jnp.float32),
    )(sums, amounts, W, b2d)


def kernel(descriptions, amounts, table, W, b):
    desc_flat4 = descriptions.reshape(-1).astype(jnp.int32) * 4
    table_rm = _tc_transpose(jnp.swapaxes(table, 0, 1)).reshape(4 * VOCAB, EMBED)
    sums = _sc_pool(table_rm, desc_flat4)
    return _tc_linear(sums, amounts, W, b.reshape(1, NUM_CAT))
